# baseline (speedup 1.0000x reference)
import functools

import jax
import jax.numpy as jnp
from jax import lax
from jax.experimental import pallas as pl
from jax.experimental.pallas import tpu as pltpu
from jax.experimental.pallas import tpu_sc as plsc

B = 4096
L = 50
D = 64
NCLS = 6

NC = 2
NS = 16
NW = NC * NS
B_PER_W = B // NW
CHUNK = 128
NCHUNK = (B_PER_W * L) // CHUNK
ROWS_PER_SC = B // NC


def _sc_pool_body(x_hbm, seg_hbm, table_hbm, zeros_hbm, out_hbm,
                  idxs, segs, rows, shared, gsem, ssem):
    c = lax.axis_index("c")
    s = lax.axis_index("s")
    wid = c * NS + s

    pltpu.sync_copy(x_hbm.at[wid], idxs)
    pltpu.sync_copy(seg_hbm.at[wid], segs)
    pltpu.sync_copy(zeros_hbm, shared.at[pl.ds(s * B_PER_W, B_PER_W)])

    NBUF = rows.shape[0]
    for g in range(min(NBUF, NCHUNK)):
        pltpu.async_copy(table_hbm.at[idxs.at[g]], rows.at[g % NBUF], gsem)
    for g in range(NCHUNK):
        pltpu.make_async_copy(
            table_hbm.at[idxs.at[g]], rows.at[g % NBUF], gsem).wait()
        pltpu.async_copy(rows.at[g % NBUF], shared.at[segs.at[g]], ssem,
                         add=True)
        if g + NBUF < NCHUNK:
            pltpu.make_async_copy(
                rows.at[g % NBUF], shared.at[segs.at[g]], ssem).wait()
            pltpu.async_copy(table_hbm.at[idxs.at[g + NBUF]],
                             rows.at[g % NBUF], gsem)
    for g in range(max(0, NCHUNK - NBUF), NCHUNK):
        pltpu.make_async_copy(
            rows.at[g % NBUF], shared.at[segs.at[g]], ssem).wait()

    pltpu.sync_copy(shared.at[pl.ds(s * B_PER_W, B_PER_W)],
                    out_hbm.at[pl.ds(wid * B_PER_W, B_PER_W)])


@functools.partial(jax.jit, static_argnames=())
def _sc_pool(x_chunks, seg_chunks, table, zeros):
    mesh = plsc.VectorSubcoreMesh(core_axis_name="c", subcore_axis_name="s")
    kern = pl.kernel(
        _sc_pool_body,
        out_type=jax.ShapeDtypeStruct((B, D), jnp.float32),
        mesh=mesh,
        scratch_types=[
            pltpu.VMEM((NCHUNK, CHUNK), jnp.int32),
            pltpu.VMEM((NCHUNK, CHUNK), jnp.int32),
            pltpu.VMEM((4, CHUNK, D), jnp.float32),
            pltpu.VMEM_SHARED((ROWS_PER_SC, D), jnp.float32),
            pltpu.SemaphoreType.DMA,
            pltpu.SemaphoreType.DMA,
        ],
        compiler_params=pltpu.CompilerParams(use_tc_tiling_on_sc=False),
    )
    return kern(x_chunks, seg_chunks, table, zeros)


def _linear_body(ps_ref, wt_ref, b_ref, o_ref):
    o_ref[...] = (
        jnp.dot(ps_ref[...], wt_ref[...], preferred_element_type=jnp.float32)
        * (1.0 / L)
        + b_ref[...]
    )


def _linear(psum, wt, b2):
    return pl.pallas_call(
        _linear_body,
        out_shape=jax.ShapeDtypeStruct((B, NCLS), jnp.float32),
    )(psum, wt, b2)


def kernel(x, table, W, b):
    x_chunks = x.astype(jnp.int32).reshape(NW, NCHUNK, CHUNK)
    seg = (jnp.arange(B * L, dtype=jnp.int32) // L) % ROWS_PER_SC
    seg_chunks = seg.reshape(NW, NCHUNK, CHUNK)
    zeros = jnp.zeros((B_PER_W, D), jnp.float32)
    psum = _sc_pool(x_chunks, seg_chunks, table, zeros)
    return _linear(psum, W.T.astype(jnp.float32), b.reshape(1, NCLS))
